# baseline (speedup 1.0000x reference)
import jax
import jax.numpy as jnp
from jax import lax
from jax.experimental import pallas as pl
from jax.experimental.pallas import tpu as pltpu
from jax.experimental.pallas import tpu_sc as plsc

B, H, S, D = 8, 4, 2048, 128
SW = 512
BH = B * H
CACHE_ROWS = BH * S
CR = 128
NCHUNK = S // CR
NBP = 2
PERIOD = 2 * NBP
L = 2


def _sc_body(k0, v0, k1, v1, lat, pos_hbm, out,
             pos_v, tbuf0, tbuf1, shared, sem2, *sems):
    s_id = lax.axis_index("s")
    w = s_id * 2 + lax.axis_index("c")
    row_lo = w * S
    tbufs = (tbuf0, tbuf1)
    sins = sems[:PERIOD]
    souts = sems[PERIOD:]

    n = 4 * NCHUNK

    def buf(i):
        slot = (i // 2) % NBP
        if i % 2 == 0:
            return tbufs[slot]
        return shared.at[s_id, slot]

    def chunk(i):
        c, j = divmod(i, NCHUNK)
        src = (k0, v0, k1, v1)[c]
        lo = row_lo + j * CR
        return (src.at[pl.ds(lo, CR)],
                out.at[pl.ds(c * CACHE_ROWS + lo, CR)])

    in_cps = [None] * n
    out_cps = [None] * n

    def start_in(i):
        if i >= PERIOD:
            out_cps[i - PERIOD].wait()
        src_sl, _ = chunk(i)
        in_cps[i] = pltpu.make_async_copy(src_sl, buf(i), sins[i % PERIOD])
        in_cps[i].start()

    def start_out(i):
        in_cps[i].wait()
        _, dst_sl = chunk(i)
        out_cps[i] = pltpu.make_async_copy(buf(i), dst_sl, souts[i % PERIOD])
        out_cps[i].start()

    for i in range(n):
        start_in(i)
        if i >= L:
            start_out(i - L)
    for i in range(n - L, n):
        start_out(i)
    for i in range(n - PERIOD, n):
        out_cps[i].wait()

    pltpu.sync_copy(pos_hbm, pos_v)
    pv = pos_v[pl.ds(w // H, 16)]
    p0 = pv[0]
    p1 = lax.bitwise_and(p0, SW - 1)

    rcps = [
        pltpu.make_async_copy(
            lat.at[pl.ds(c * BH + w, 1)],
            out.at[pl.ds(c * CACHE_ROWS + row_lo + (p0 if c < 2 else p1), 1)],
            sem2,
        )
        for c in range(4)
    ]
    for cp in rcps:
        cp.start()
    for cp in rcps:
        cp.wait()


def kernel(k_cache_0, v_cache_0, k_cache_1, v_cache_1,
           latest_k_0, latest_v_0, latest_k_1, latest_v_1, position_ids):
    caches = [cc.reshape(BH * S, D)
              for cc in (k_cache_0, v_cache_0, k_cache_1, v_cache_1)]
    lat = jnp.stack([latest_k_0, latest_v_0, latest_k_1, latest_v_1],
                    axis=0).reshape(4 * BH, D)
    pos = jnp.pad(position_ids.reshape(B).astype(jnp.int32), (0, 16))

    mesh = plsc.VectorSubcoreMesh(core_axis_name="c", subcore_axis_name="s")
    run = pl.kernel(
        _sc_body,
        out_type=jax.ShapeDtypeStruct((4 * CACHE_ROWS, D), jnp.float32),
        mesh=mesh,
        scratch_types=[
            pltpu.VMEM((24,), jnp.int32),
            pltpu.VMEM((CR, D), jnp.float32),
            pltpu.VMEM((CR, D), jnp.float32),
            pltpu.MemorySpace.VMEM_SHARED((16, NBP, CR, D), jnp.float32),
            pltpu.SemaphoreType.DMA,
        ] + [pltpu.SemaphoreType.DMA] * (2 * PERIOD),
    )
    out = run(*caches, lat, pos)
    return out.reshape(4, B, H, S, D)


# speedup vs baseline: 1.0839x; 1.0839x over previous
import jax
import jax.numpy as jnp
from jax import lax
from jax.experimental import pallas as pl
from jax.experimental.pallas import tpu as pltpu
from jax.experimental.pallas import tpu_sc as plsc

B, H, S, D = 8, 4, 2048, 128
SW = 512
BH = B * H
CACHE_ROWS = BH * S
SLABS = 2
TS = SLABS * S


def _tc_copy_body(k0, v0, k1, v1, out):
    out[0] = k0[...]
    out[1] = v0[...]
    out[2] = k1[...]
    out[3] = v1[...]


def _sc_patch(out_ref, lat_ref, pos_ref):
    mesh = plsc.ScalarSubcoreMesh(axis_name="c", num_cores=2)

    @pl.core_map(
        mesh,
        scratch_shapes=[pltpu.SMEM((24,), jnp.int32), pltpu.SemaphoreType.DMA],
    )
    def _(pos_s, sem):
        core = lax.axis_index("c")

        pltpu.sync_copy(pos_ref, pos_s)
        rcps = []
        for j in range(16):
            w = core * 16 + j
            p0 = pos_s[w // H]
            p1 = lax.bitwise_and(p0, SW - 1)
            row_lo = w * S
            for c in range(4):
                rcps.append(pltpu.make_async_copy(
                    lat_ref.at[pl.ds(c * BH + w, 1)],
                    out_ref.at[pl.ds(
                        c * CACHE_ROWS + row_lo + (p0 if c < 2 else p1), 1)],
                    sem,
                ))
        for cp in rcps:
            cp.start()
        for cp in rcps:
            cp.wait()


def kernel(k_cache_0, v_cache_0, k_cache_1, v_cache_1,
           latest_k_0, latest_v_0, latest_k_1, latest_v_1, position_ids):
    caches = [cc.reshape(BH * S, D)
              for cc in (k_cache_0, v_cache_0, k_cache_1, v_cache_1)]
    lat = jnp.stack([latest_k_0, latest_v_0, latest_k_1, latest_v_1],
                    axis=0).reshape(4 * BH, D)
    pos = jnp.pad(position_ids.reshape(B).astype(jnp.int32), (0, 16))

    out0 = pl.pallas_call(
        _tc_copy_body,
        grid=(BH // SLABS,),
        in_specs=[pl.BlockSpec((TS, D), lambda t: (t, 0))] * 4,
        out_specs=pl.BlockSpec((4, TS, D), lambda t: (0, t, 0)),
        out_shape=jax.ShapeDtypeStruct((4, CACHE_ROWS, D), jnp.float32),
    )(*caches)
    out0 = out0.reshape(4 * CACHE_ROWS, D)

    def upd(refs):
        out_ref, lat_ref, pos_ref = refs
        _sc_patch(out_ref, lat_ref, pos_ref)

    out1, _, _ = pl.run_state(upd)((out0, lat, pos))
    return out1.reshape(4, B, H, S, D)
